# Initial kernel scaffold; baseline (speedup 1.0000x reference)
import jax
import jax.numpy as jnp
from jax.experimental import pallas as pl


def kernel(inputs, gate_kernel):
    raise NotImplementedError("write your pallas kernel here")


# speedup vs baseline: 1.1061x; 1.1061x over previous
import functools

import jax
import jax.numpy as jnp
from jax.experimental import pallas as pl
from jax.experimental.pallas import tpu as pltpu

NUM_EXPERTS = 64
TOP_K = 8
D_MODEL = 4096
N_TOKENS = 32768
BLOCK_T = 512


def _router_block(x_ref, w_ref, ew_ref, idx_ref, logits_ref, probs_ref):
    logits = jnp.dot(x_ref[...], w_ref[...], preferred_element_type=jnp.float32)
    logits_ref[...] = logits

    iota_e = jax.lax.broadcasted_iota(jnp.int32, logits.shape, 1)
    work = logits
    mask = jnp.zeros(logits.shape, dtype=jnp.bool_)
    idx_cols = []
    for _ in range(TOP_K):
        m = jnp.max(work, axis=1, keepdims=True)
        cand = jnp.where(work == m, iota_e, NUM_EXPERTS)
        idx_k = jnp.min(cand, axis=1, keepdims=True)
        sel = iota_e == idx_k
        mask = jnp.logical_or(mask, sel)
        work = jnp.where(sel, -jnp.inf, work)
        idx_cols.append(idx_k)
    idx_ref[...] = jnp.concatenate(idx_cols, axis=1)

    m0 = jnp.max(logits, axis=1, keepdims=True)
    p = jnp.exp(logits - m0)
    probs_ref[...] = p / jnp.sum(p, axis=1, keepdims=True)
    p_sel = jnp.where(mask, p, 0.0)
    ew_ref[...] = p_sel / jnp.sum(p_sel, axis=1, keepdims=True)


@jax.jit
def kernel(inputs, gate_kernel):
    n_tokens, d_model = inputs.shape
    grid = (n_tokens // BLOCK_T,)
    out_shapes = (
        jax.ShapeDtypeStruct((n_tokens, NUM_EXPERTS), jnp.float32),
        jax.ShapeDtypeStruct((n_tokens, TOP_K), jnp.int32),
        jax.ShapeDtypeStruct((n_tokens, NUM_EXPERTS), jnp.float32),
        jax.ShapeDtypeStruct((n_tokens, NUM_EXPERTS), jnp.float32),
    )
    tok_spec = lambda w: pl.BlockSpec((BLOCK_T, w), lambda i: (i, 0))
    out = pl.pallas_call(
        _router_block,
        grid=grid,
        in_specs=[
            pl.BlockSpec((BLOCK_T, d_model), lambda i: (i, 0)),
            pl.BlockSpec((d_model, NUM_EXPERTS), lambda i: (0, 0)),
        ],
        out_specs=(
            tok_spec(NUM_EXPERTS),
            tok_spec(TOP_K),
            tok_spec(NUM_EXPERTS),
            tok_spec(NUM_EXPERTS),
        ),
        out_shape=out_shapes,
        compiler_params=pltpu.CompilerParams(
            dimension_semantics=("arbitrary",),
        ),
    )(inputs, gate_kernel)
    return out


# speedup vs baseline: 1.2538x; 1.1335x over previous
import functools

import jax
import jax.numpy as jnp
from jax.experimental import pallas as pl
from jax.experimental.pallas import tpu as pltpu

NUM_EXPERTS = 64
TOP_K = 8
D_MODEL = 4096
N_TOKENS = 32768
BLOCK_T = 1024


def _router_block(x_ref, w_ref, ew_ref, idx_ref, logits_ref, probs_ref):
    logits = jnp.dot(x_ref[...], w_ref[...], preferred_element_type=jnp.float32)
    logits_ref[...] = logits

    iota_e = jax.lax.broadcasted_iota(jnp.int32, logits.shape, 1)
    work = logits
    mask = jnp.zeros(logits.shape, dtype=jnp.bool_)
    idx_cols = []
    for _ in range(TOP_K):
        m = jnp.max(work, axis=1, keepdims=True)
        cand = jnp.where(work == m, iota_e, NUM_EXPERTS)
        idx_k = jnp.min(cand, axis=1, keepdims=True)
        sel = iota_e == idx_k
        mask = jnp.logical_or(mask, sel)
        work = jnp.where(sel, -jnp.inf, work)
        idx_cols.append(idx_k)
    idx_ref[...] = jnp.concatenate(idx_cols, axis=1)

    m0 = jnp.max(logits, axis=1, keepdims=True)
    p = jnp.exp(logits - m0)
    probs_ref[...] = p / jnp.sum(p, axis=1, keepdims=True)
    p_sel = jnp.where(mask, p, 0.0)
    ew_ref[...] = p_sel / jnp.sum(p_sel, axis=1, keepdims=True)


@jax.jit
def kernel(inputs, gate_kernel):
    n_tokens, d_model = inputs.shape
    grid = (n_tokens // BLOCK_T,)
    out_shapes = (
        jax.ShapeDtypeStruct((n_tokens, NUM_EXPERTS), jnp.float32),
        jax.ShapeDtypeStruct((n_tokens, TOP_K), jnp.int32),
        jax.ShapeDtypeStruct((n_tokens, NUM_EXPERTS), jnp.float32),
        jax.ShapeDtypeStruct((n_tokens, NUM_EXPERTS), jnp.float32),
    )
    tok_spec = lambda w: pl.BlockSpec((BLOCK_T, w), lambda i: (i, 0))
    out = pl.pallas_call(
        _router_block,
        grid=grid,
        in_specs=[
            pl.BlockSpec((BLOCK_T, d_model), lambda i: (i, 0)),
            pl.BlockSpec((d_model, NUM_EXPERTS), lambda i: (0, 0)),
        ],
        out_specs=(
            tok_spec(NUM_EXPERTS),
            tok_spec(TOP_K),
            tok_spec(NUM_EXPERTS),
            tok_spec(NUM_EXPERTS),
        ),
        out_shape=out_shapes,
        compiler_params=pltpu.CompilerParams(
            dimension_semantics=("arbitrary",),
        ),
    )(inputs, gate_kernel)
    return out
